# baseline (speedup 1.0000x reference)
import functools

import jax
import jax.numpy as jnp
from jax import lax
from jax.experimental import pallas as pl
from jax.experimental.pallas import tpu as pltpu
from jax.experimental.pallas import tpu_sc as plsc

B = 4096
EMB = 64
SEQ = 32
IMG_D = 512
N_ITEMS = 100001
N_TOKENS = 20000

NC = 2
NS = 16
NW = NC * NS
BPW = B // NW
TOK_PER_W = BPW * SEQ
CHUNK = 128
NCH = TOK_PER_W // CHUNK
ROWS_PER_CHUNK = CHUNK // SEQ
NL = EMB // 16

PACK_BLK = 1024


def _pack_body(left_ref, right_ref, out_ref):
  ident = jnp.eye(EMB, dtype=jnp.float32)
  dn = (((0,), (0,)), ((), ()))
  lt = jax.lax.dot_general(left_ref[...], ident, dn,
                           preferred_element_type=jnp.float32)
  rt = jax.lax.dot_general(right_ref[...], ident, dn,
                           preferred_element_type=jnp.float32)
  out_ref[...] = jnp.concatenate([lt, rt], axis=1)


def _pack_table(table, n_rows):
  nblk = (n_rows + PACK_BLK - 1) // PACK_BLK
  assert nblk % 2 == 0, "even block count required"
  pad = nblk * PACK_BLK
  half_blk = nblk // 2
  t_t = table.T
  out = pl.pallas_call(
      _pack_body,
      grid=(half_blk,),
      in_specs=[
          pl.BlockSpec((EMB, PACK_BLK), lambda i: (0, i)),
          pl.BlockSpec((EMB, PACK_BLK), lambda i: (0, half_blk + i)),
      ],
      out_specs=pl.BlockSpec((PACK_BLK, 2 * EMB), lambda i: (i, 0)),
      out_shape=jax.ShapeDtypeStruct((pad // 2, 2 * EMB), jnp.float32),
  )(t_t, t_t)
  return out, pad // 2


def _remap_idx(idx, half):
  return jnp.where(idx < half, 2 * idx, 2 * (idx - half) + 1).astype(jnp.int32)


def _sc_text(tok2d, text_packed):
  rows2 = text_packed.shape[0] * 2
  text_lin = text_packed.reshape(rows2, EMB)
  mesh = plsc.VectorSubcoreMesh(core_axis_name="c", subcore_axis_name="s")

  @functools.partial(
      pl.kernel,
      mesh=mesh,
      compiler_params=pltpu.CompilerParams(use_tc_tiling_on_sc=False),
      out_type=jax.ShapeDtypeStruct((B, 2 * EMB), jnp.float32),
      scratch_types=[
          pltpu.VMEM((NCH, CHUNK), jnp.int32),
          pltpu.VMEM((CHUNK, EMB), jnp.float32),
          pltpu.VMEM((CHUNK, EMB), jnp.float32),
          pltpu.VMEM((BPW, 2 * EMB), jnp.float32),
          pltpu.SemaphoreType.DMA,
          pltpu.SemaphoreType.DMA,
      ],
  )
  def sc_kernel(tok_hbm, table_hbm, text_out_hbm,
                tok_idx_v, gbuf0, gbuf1, acc, sem_t0, sem_t1):
    wid = lax.axis_index("s") * NC + lax.axis_index("c")
    base = wid * BPW
    gbufs = (gbuf0, gbuf1)
    sems = (sem_t0, sem_t1)

    pltpu.sync_copy(tok_hbm.at[pl.ds(wid * NCH, NCH)], tok_idx_v)
    pltpu.async_copy(table_hbm.at[tok_idx_v.at[0]], gbufs[0], sems[0])

    @pl.loop(0, NCH, step=2)
    def chunk_loop(c):
      for b in range(2):
        cc = c + b
        nxt = (b + 1) % 2

        @pl.when(cc + 1 < NCH)
        def _():
          pltpu.async_copy(table_hbm.at[tok_idx_v.at[cc + 1]],
                           gbufs[nxt], sems[nxt])

        pltpu.make_async_copy(table_hbm.at[tok_idx_v.at[0]],
                              gbufs[b], sems[b]).wait()
        gbuf = gbufs[b]
        for i in range(ROWS_PER_CHUNK):
          accs = [None] * NL
          for j in range(SEQ):
            r = i * SEQ + j
            for l in range(NL):
              v = gbuf[r, pl.ds(l * 16, 16)]
              accs[l] = v if accs[l] is None else accs[l] + v
          row = cc * ROWS_PER_CHUNK + i
          for l in range(NL):
            acc[row, pl.ds(l * 16, 16)] = accs[l]

    pltpu.sync_copy(acc, text_out_hbm.at[pl.ds(base, BPW)])

  return sc_kernel(tok2d, text_lin)


IDX_ROWS = B * EMB // 128
IDX_RPW = IDX_ROWS // NW


def _sc_item(idx2d, table_1d):
  mesh = plsc.VectorSubcoreMesh(core_axis_name="c", subcore_axis_name="s")

  @functools.partial(
      pl.kernel,
      mesh=mesh,
      compiler_params=pltpu.CompilerParams(use_tc_tiling_on_sc=False),
      out_type=jax.ShapeDtypeStruct((IDX_ROWS, 128), jnp.float32),
      scratch_types=[
          pltpu.VMEM((IDX_RPW, 128), jnp.int32),
          pltpu.VMEM((IDX_RPW, 128), jnp.float32),
          pltpu.SemaphoreType.DMA,
      ],
  )
  def sc_kernel(idx_hbm, table_hbm, out_hbm, idx_v, dst_v, sem):
    wid = lax.axis_index("s") * NC + lax.axis_index("c")
    base = wid * IDX_RPW
    pltpu.sync_copy(idx_hbm.at[pl.ds(base, IDX_RPW)], idx_v)
    cps = [pltpu.async_copy(table_hbm.at[idx_v.at[j]], dst_v.at[j], sem)
           for j in range(IDX_RPW)]
    for cp in cps:
      cp.wait()
    pltpu.sync_copy(dst_v, out_hbm.at[pl.ds(base, IDX_RPW)])

  return sc_kernel(idx2d, table_1d)


def _tc_img_body(img_ref, wimg_ref, bimg_ref, out_ref):
  out_ref[...] = jnp.dot(img_ref[...], wimg_ref[...],
                         preferred_element_type=jnp.float32) + bimg_ref[...]


def _tc_img(image_embedding, W_img, b_img):
  TILE = 512
  return pl.pallas_call(
      _tc_img_body,
      grid=(B // TILE,),
      in_specs=[
          pl.BlockSpec((TILE, IMG_D), lambda i: (i, 0)),
          pl.BlockSpec((IMG_D, EMB), lambda i: (0, 0)),
          pl.BlockSpec((1, EMB), lambda i: (0, 0)),
      ],
      out_specs=pl.BlockSpec((TILE, EMB), lambda i: (i, 0)),
      out_shape=jax.ShapeDtypeStruct((B, EMB), jnp.float32),
  )(image_embedding, W_img, b_img)


def _tc_final_body(item_ref, text_ref, img_ref, wproj_ref, bproj_ref,
                   out_ref):
  w = wproj_ref[...]
  part = jnp.dot(item_ref[...], w[:EMB, :],
                 preferred_element_type=jnp.float32)
  part += jnp.dot(text_ref[:, :EMB] * (1.0 / SEQ), w[EMB:2 * EMB, :],
                  preferred_element_type=jnp.float32)
  part += jnp.dot(img_ref[...], w[2 * EMB:, :],
                  preferred_element_type=jnp.float32)
  out_ref[...] = part + bproj_ref[...]


def _tc_final(item_vec, text_sum, img_vec, W_proj, b_proj):
  TILE = 512
  return pl.pallas_call(
      _tc_final_body,
      grid=(B // TILE,),
      in_specs=[
          pl.BlockSpec((TILE, EMB), lambda i: (i, 0)),
          pl.BlockSpec((TILE, 2 * EMB), lambda i: (i, 0)),
          pl.BlockSpec((TILE, EMB), lambda i: (i, 0)),
          pl.BlockSpec((3 * EMB, EMB), lambda i: (0, 0)),
          pl.BlockSpec((1, EMB), lambda i: (0, 0)),
      ],
      out_specs=pl.BlockSpec((TILE, EMB), lambda i: (i, 0)),
      out_shape=jax.ShapeDtypeStruct((B, EMB), jnp.float32),
  )(item_vec, text_sum, img_vec, W_proj, b_proj)


@jax.jit
def kernel(item_id, text_tokens, image_embedding, item_table, text_table,
           W_img, b_img, W_proj, b_proj):
  text_packed, text_half = _pack_table(text_table, N_TOKENS)
  tok_mapped = _remap_idx(text_tokens, text_half)
  tok2d = tok_mapped.reshape(B * SEQ // CHUNK, CHUNK)
  text_sum = _sc_text(tok2d, text_packed)

  table_1d = item_table.T.reshape(N_ITEMS * EMB)
  offs = (jnp.arange(EMB, dtype=jnp.int32) * N_ITEMS)[None, :]
  idx2d = (item_id[:, None] + offs).reshape(IDX_ROWS, 128)
  item_vec = _sc_item(idx2d, table_1d).reshape(B, EMB)

  img_vec = _tc_img(image_embedding, W_img, b_img.reshape(1, EMB))
  return _tc_final(item_vec, text_sum, img_vec, W_proj,
                   b_proj.reshape(1, EMB))


# speedup vs baseline: 3.4059x; 3.4059x over previous
import functools

import jax
import jax.numpy as jnp
from jax import lax
from jax.experimental import pallas as pl
from jax.experimental.pallas import tpu as pltpu
from jax.experimental.pallas import tpu_sc as plsc

B = 4096
EMB = 64
SEQ = 32
IMG_D = 512
N_ITEMS = 100001
N_TOKENS = 20000

NC = 2
NS = 16
NW = NC * NS
BPW = B // NW
TOK_PER_W = BPW * SEQ
CHUNK = 128
NCH = TOK_PER_W // CHUNK
ROWS_PER_CHUNK = CHUNK // SEQ
NL = EMB // 16

PACK_BLK = 1024


def _pack_body(left_ref, right_ref, out_ref):
  ident = jnp.eye(EMB, dtype=jnp.float32)
  dn = (((0,), (0,)), ((), ()))
  lt = jax.lax.dot_general(left_ref[...], ident, dn,
                           preferred_element_type=jnp.float32)
  rt = jax.lax.dot_general(right_ref[...], ident, dn,
                           preferred_element_type=jnp.float32)
  out_ref[...] = jnp.concatenate([lt, rt], axis=1)


def _pack_table(table, n_rows):
  nblk = (n_rows + PACK_BLK - 1) // PACK_BLK
  assert nblk % 2 == 0, "even block count required"
  pad = nblk * PACK_BLK
  half_blk = nblk // 2
  t_t = table.T
  out = pl.pallas_call(
      _pack_body,
      grid=(half_blk,),
      in_specs=[
          pl.BlockSpec((EMB, PACK_BLK), lambda i: (0, i)),
          pl.BlockSpec((EMB, PACK_BLK), lambda i: (0, half_blk + i)),
      ],
      out_specs=pl.BlockSpec((PACK_BLK, 2 * EMB), lambda i: (i, 0)),
      out_shape=jax.ShapeDtypeStruct((pad // 2, 2 * EMB), jnp.float32),
  )(t_t, t_t)
  return out, pad // 2


def _remap_idx(idx, half):
  return jnp.where(idx < half, 2 * idx, 2 * (idx - half) + 1).astype(jnp.int32)


def _sc_text(tok2d, text_packed):
  rows2 = text_packed.shape[0] * 2
  text_lin = text_packed.reshape(rows2, EMB)
  mesh = plsc.VectorSubcoreMesh(core_axis_name="c", subcore_axis_name="s")

  @functools.partial(
      pl.kernel,
      mesh=mesh,
      compiler_params=pltpu.CompilerParams(use_tc_tiling_on_sc=False),
      out_type=jax.ShapeDtypeStruct((B, 2 * EMB), jnp.float32),
      scratch_types=[
          pltpu.VMEM((NCH, CHUNK), jnp.int32),
          pltpu.VMEM((CHUNK, EMB), jnp.float32),
          pltpu.VMEM((CHUNK, EMB), jnp.float32),
          pltpu.VMEM((BPW, 2 * EMB), jnp.float32),
          pltpu.SemaphoreType.DMA,
          pltpu.SemaphoreType.DMA,
      ],
  )
  def sc_kernel(tok_hbm, table_hbm, text_out_hbm,
                tok_idx_v, gbuf0, gbuf1, acc, sem_t0, sem_t1):
    wid = lax.axis_index("s") * NC + lax.axis_index("c")
    base = wid * BPW
    gbufs = (gbuf0, gbuf1)
    sems = (sem_t0, sem_t1)

    pltpu.sync_copy(tok_hbm.at[pl.ds(wid * NCH, NCH)], tok_idx_v)
    pltpu.async_copy(table_hbm.at[tok_idx_v.at[0]], gbufs[0], sems[0])

    @pl.loop(0, NCH, step=2)
    def chunk_loop(c):
      for b in range(2):
        cc = c + b
        nxt = (b + 1) % 2

        @pl.when(cc + 1 < NCH)
        def _():
          pltpu.async_copy(table_hbm.at[tok_idx_v.at[cc + 1]],
                           gbufs[nxt], sems[nxt])

        pltpu.make_async_copy(table_hbm.at[tok_idx_v.at[0]],
                              gbufs[b], sems[b]).wait()
        gbuf = gbufs[b]
        for i in range(ROWS_PER_CHUNK):
          accs = [None] * NL
          for j in range(SEQ):
            r = i * SEQ + j
            for l in range(NL):
              v = gbuf[r, pl.ds(l * 16, 16)]
              accs[l] = v if accs[l] is None else accs[l] + v
          row = cc * ROWS_PER_CHUNK + i
          for l in range(NL):
            acc[row, pl.ds(l * 16, 16)] = accs[l]

    pltpu.sync_copy(acc, text_out_hbm.at[pl.ds(base, BPW)])

  return sc_kernel(tok2d, text_lin)


def _sc_item(item_idx, item_pad):
  mesh = plsc.VectorSubcoreMesh(core_axis_name="c", subcore_axis_name="s")

  @functools.partial(
      pl.kernel,
      mesh=mesh,
      compiler_params=pltpu.CompilerParams(use_tc_tiling_on_sc=False),
      out_type=jax.ShapeDtypeStruct((B, 2 * EMB), jnp.float32),
      scratch_types=[
          pltpu.VMEM((BPW,), jnp.int32),
          pltpu.VMEM((BPW, 2 * EMB), jnp.float32),
          pltpu.SemaphoreType.DMA,
      ],
  )
  def sc_kernel(idx_hbm, table_hbm, out_hbm, idx_v, rows_v, sem):
    wid = lax.axis_index("s") * NC + lax.axis_index("c")
    base = wid * BPW
    pltpu.sync_copy(idx_hbm.at[pl.ds(base, BPW)], idx_v)
    pltpu.async_copy(table_hbm.at[idx_v], rows_v, sem).wait()
    pltpu.sync_copy(rows_v, out_hbm.at[pl.ds(base, BPW)])

  return sc_kernel(item_idx, item_pad)


def _tc_img_body(img_ref, wimg_ref, bimg_ref, out_ref):
  out_ref[...] = jnp.dot(img_ref[...], wimg_ref[...],
                         preferred_element_type=jnp.float32) + bimg_ref[...]


def _tc_img(image_embedding, W_img, b_img):
  TILE = 512
  return pl.pallas_call(
      _tc_img_body,
      grid=(B // TILE,),
      in_specs=[
          pl.BlockSpec((TILE, IMG_D), lambda i: (i, 0)),
          pl.BlockSpec((IMG_D, EMB), lambda i: (0, 0)),
          pl.BlockSpec((1, EMB), lambda i: (0, 0)),
      ],
      out_specs=pl.BlockSpec((TILE, EMB), lambda i: (i, 0)),
      out_shape=jax.ShapeDtypeStruct((B, EMB), jnp.float32),
  )(image_embedding, W_img, b_img)


def _tc_final_body(item_ref, text_ref, img_ref, wproj_ref, bproj_ref,
                   out_ref):
  w = wproj_ref[...]
  part = jnp.dot(item_ref[:, :EMB], w[:EMB, :],
                 preferred_element_type=jnp.float32)
  part += jnp.dot(text_ref[:, :EMB] * (1.0 / SEQ), w[EMB:2 * EMB, :],
                  preferred_element_type=jnp.float32)
  part += jnp.dot(img_ref[...], w[2 * EMB:, :],
                  preferred_element_type=jnp.float32)
  out_ref[...] = part + bproj_ref[...]


def _tc_final(item_vec, text_sum, img_vec, W_proj, b_proj):
  TILE = 512
  return pl.pallas_call(
      _tc_final_body,
      grid=(B // TILE,),
      in_specs=[
          pl.BlockSpec((TILE, 2 * EMB), lambda i: (i, 0)),
          pl.BlockSpec((TILE, 2 * EMB), lambda i: (i, 0)),
          pl.BlockSpec((TILE, EMB), lambda i: (i, 0)),
          pl.BlockSpec((3 * EMB, EMB), lambda i: (0, 0)),
          pl.BlockSpec((1, EMB), lambda i: (0, 0)),
      ],
      out_specs=pl.BlockSpec((TILE, EMB), lambda i: (i, 0)),
      out_shape=jax.ShapeDtypeStruct((B, EMB), jnp.float32),
  )(item_vec, text_sum, img_vec, W_proj, b_proj)


@jax.jit
def kernel(item_id, text_tokens, image_embedding, item_table, text_table,
           W_img, b_img, W_proj, b_proj):
  text_packed, text_half = _pack_table(text_table, N_TOKENS)
  tok_mapped = _remap_idx(text_tokens, text_half)
  tok2d = tok_mapped.reshape(B * SEQ // CHUNK, CHUNK)
  text_sum = _sc_text(tok2d, text_packed)

  item_pad = jnp.pad(item_table, ((0, 7), (0, EMB)))
  item_vec = _sc_item(item_id, item_pad)

  img_vec = _tc_img(image_embedding, W_img, b_img.reshape(1, EMB))
  return _tc_final(item_vec, text_sum, img_vec, W_proj,
                   b_proj.reshape(1, EMB))


# speedup vs baseline: 3.4146x; 1.0026x over previous
import functools

import jax
import jax.numpy as jnp
from jax import lax
from jax.experimental import pallas as pl
from jax.experimental.pallas import tpu as pltpu
from jax.experimental.pallas import tpu_sc as plsc

B = 4096
EMB = 64
SEQ = 32
IMG_D = 512
N_ITEMS = 100001
N_TOKENS = 20000

NC = 2
NS = 16
NW = NC * NS
BPW = B // NW
TOK_PER_W = BPW * SEQ
CHUNK = 128
NCH = TOK_PER_W // CHUNK
ROWS_PER_CHUNK = CHUNK // SEQ
NL = EMB // 16

PACK_BLK = 1024


def _pack_body(left_ref, right_ref, out_ref):
  out_ref[...] = jnp.concatenate([left_ref[...].T, right_ref[...].T], axis=1)


def _pack_table(table, n_rows):
  nblk = (n_rows + PACK_BLK - 1) // PACK_BLK
  assert nblk % 2 == 0, "even block count required"
  pad = nblk * PACK_BLK
  half_blk = nblk // 2
  t_t = table.T
  out = pl.pallas_call(
      _pack_body,
      grid=(half_blk,),
      in_specs=[
          pl.BlockSpec((EMB, PACK_BLK), lambda i: (0, i)),
          pl.BlockSpec((EMB, PACK_BLK), lambda i: (0, half_blk + i)),
      ],
      out_specs=pl.BlockSpec((PACK_BLK, 2 * EMB), lambda i: (i, 0)),
      out_shape=jax.ShapeDtypeStruct((pad // 2, 2 * EMB), jnp.float32),
  )(t_t, t_t)
  return out, pad // 2


def _remap_idx(idx, half):
  return jnp.where(idx < half, 2 * idx, 2 * (idx - half) + 1).astype(jnp.int32)


def _sc_text(tok2d, text_packed):
  rows2 = text_packed.shape[0] * 2
  text_lin = text_packed.reshape(rows2, EMB)
  mesh = plsc.VectorSubcoreMesh(core_axis_name="c", subcore_axis_name="s")

  @functools.partial(
      pl.kernel,
      mesh=mesh,
      compiler_params=pltpu.CompilerParams(use_tc_tiling_on_sc=False),
      out_type=jax.ShapeDtypeStruct((B, 2 * EMB), jnp.float32),
      scratch_types=[
          pltpu.VMEM((NCH, CHUNK), jnp.int32),
          pltpu.VMEM((CHUNK, EMB), jnp.float32),
          pltpu.VMEM((CHUNK, EMB), jnp.float32),
          pltpu.VMEM((BPW, 2 * EMB), jnp.float32),
          pltpu.SemaphoreType.DMA,
          pltpu.SemaphoreType.DMA,
      ],
  )
  def sc_kernel(tok_hbm, table_hbm, text_out_hbm,
                tok_idx_v, gbuf0, gbuf1, acc, sem_t0, sem_t1):
    wid = lax.axis_index("s") * NC + lax.axis_index("c")
    base = wid * BPW
    gbufs = (gbuf0, gbuf1)
    sems = (sem_t0, sem_t1)

    pltpu.sync_copy(tok_hbm.at[pl.ds(wid * NCH, NCH)], tok_idx_v)
    pltpu.async_copy(table_hbm.at[tok_idx_v.at[0]], gbufs[0], sems[0])

    @pl.loop(0, NCH, step=2)
    def chunk_loop(c):
      for b in range(2):
        cc = c + b
        nxt = (b + 1) % 2

        @pl.when(cc + 1 < NCH)
        def _():
          pltpu.async_copy(table_hbm.at[tok_idx_v.at[cc + 1]],
                           gbufs[nxt], sems[nxt])

        pltpu.make_async_copy(table_hbm.at[tok_idx_v.at[0]],
                              gbufs[b], sems[b]).wait()
        gbuf = gbufs[b]
        for i in range(ROWS_PER_CHUNK):
          accs = [None] * NL
          for j in range(SEQ):
            r = i * SEQ + j
            for l in range(NL):
              v = gbuf[r, pl.ds(l * 16, 16)]
              accs[l] = v if accs[l] is None else accs[l] + v
          row = cc * ROWS_PER_CHUNK + i
          for l in range(NL):
            acc[row, pl.ds(l * 16, 16)] = accs[l]

    pltpu.sync_copy(acc, text_out_hbm.at[pl.ds(base, BPW)])

  return sc_kernel(tok2d, text_lin)


def _sc_item(item_idx, item_pad):
  mesh = plsc.VectorSubcoreMesh(core_axis_name="c", subcore_axis_name="s")

  @functools.partial(
      pl.kernel,
      mesh=mesh,
      compiler_params=pltpu.CompilerParams(use_tc_tiling_on_sc=False),
      out_type=jax.ShapeDtypeStruct((B, 2 * EMB), jnp.float32),
      scratch_types=[
          pltpu.VMEM((BPW,), jnp.int32),
          pltpu.VMEM((BPW, 2 * EMB), jnp.float32),
          pltpu.SemaphoreType.DMA,
      ],
  )
  def sc_kernel(idx_hbm, table_hbm, out_hbm, idx_v, rows_v, sem):
    wid = lax.axis_index("s") * NC + lax.axis_index("c")
    base = wid * BPW
    pltpu.sync_copy(idx_hbm.at[pl.ds(base, BPW)], idx_v)
    pltpu.async_copy(table_hbm.at[idx_v], rows_v, sem).wait()
    pltpu.sync_copy(rows_v, out_hbm.at[pl.ds(base, BPW)])

  return sc_kernel(item_idx, item_pad)


def _tc_img_body(img_ref, wimg_ref, bimg_ref, out_ref):
  out_ref[...] = jnp.dot(img_ref[...], wimg_ref[...],
                         preferred_element_type=jnp.float32) + bimg_ref[...]


def _tc_img(image_embedding, W_img, b_img):
  TILE = 512
  return pl.pallas_call(
      _tc_img_body,
      grid=(B // TILE,),
      in_specs=[
          pl.BlockSpec((TILE, IMG_D), lambda i: (i, 0)),
          pl.BlockSpec((IMG_D, EMB), lambda i: (0, 0)),
          pl.BlockSpec((1, EMB), lambda i: (0, 0)),
      ],
      out_specs=pl.BlockSpec((TILE, EMB), lambda i: (i, 0)),
      out_shape=jax.ShapeDtypeStruct((B, EMB), jnp.float32),
  )(image_embedding, W_img, b_img)


def _tc_final_body(item_ref, text_ref, img_ref, wproj_ref, bproj_ref,
                   out_ref):
  w = wproj_ref[...]
  part = jnp.dot(item_ref[:, :EMB], w[:EMB, :],
                 preferred_element_type=jnp.float32)
  part += jnp.dot(text_ref[:, :EMB] * (1.0 / SEQ), w[EMB:2 * EMB, :],
                  preferred_element_type=jnp.float32)
  part += jnp.dot(img_ref[...], w[2 * EMB:, :],
                  preferred_element_type=jnp.float32)
  out_ref[...] = part + bproj_ref[...]


def _tc_final(item_vec, text_sum, img_vec, W_proj, b_proj):
  TILE = 512
  return pl.pallas_call(
      _tc_final_body,
      grid=(B // TILE,),
      in_specs=[
          pl.BlockSpec((TILE, 2 * EMB), lambda i: (i, 0)),
          pl.BlockSpec((TILE, 2 * EMB), lambda i: (i, 0)),
          pl.BlockSpec((TILE, EMB), lambda i: (i, 0)),
          pl.BlockSpec((3 * EMB, EMB), lambda i: (0, 0)),
          pl.BlockSpec((1, EMB), lambda i: (0, 0)),
      ],
      out_specs=pl.BlockSpec((TILE, EMB), lambda i: (i, 0)),
      out_shape=jax.ShapeDtypeStruct((B, EMB), jnp.float32),
  )(item_vec, text_sum, img_vec, W_proj, b_proj)


@jax.jit
def kernel(item_id, text_tokens, image_embedding, item_table, text_table,
           W_img, b_img, W_proj, b_proj):
  text_packed, text_half = _pack_table(text_table, N_TOKENS)
  tok_mapped = _remap_idx(text_tokens, text_half)
  tok2d = tok_mapped.reshape(B * SEQ // CHUNK, CHUNK)
  text_sum = _sc_text(tok2d, text_packed)

  item_pad = jnp.pad(item_table, ((0, 7), (0, EMB)))
  item_vec = _sc_item(item_id, item_pad)

  img_vec = _tc_img(image_embedding, W_img, b_img.reshape(1, EMB))
  return _tc_final(item_vec, text_sum, img_vec, W_proj,
                   b_proj.reshape(1, EMB))


# speedup vs baseline: 3.5997x; 1.0542x over previous
import functools

import jax
import jax.numpy as jnp
from jax import lax
from jax.experimental import pallas as pl
from jax.experimental.pallas import tpu as pltpu
from jax.experimental.pallas import tpu_sc as plsc

B = 4096
EMB = 64
SEQ = 32
IMG_D = 512
N_ITEMS = 100001
N_TOKENS = 20000

NC = 2
NS = 16
NW = NC * NS
BPW = B // NW
TOK_PER_W = BPW * SEQ
CHUNK = 128
NCH = TOK_PER_W // CHUNK
ROWS_PER_CHUNK = CHUNK // SEQ
NL = EMB // 16

PACK_BLK = 1024


def _pack_body(left_ref, right_ref, out_ref):
  out_ref[...] = jnp.concatenate([left_ref[...].T, right_ref[...].T], axis=1)


def _bf16_word_pack(t):
  lo = jax.lax.bitcast_convert_type(t[:, :32].astype(jnp.bfloat16),
                                    jnp.uint16).astype(jnp.uint32)
  hi = jax.lax.bitcast_convert_type(t[:, 32:].astype(jnp.bfloat16),
                                    jnp.uint16).astype(jnp.uint32)
  return jax.lax.bitcast_convert_type(lo | (hi << 16), jnp.float32)


def _pack_text_body(q0_ref, q1_ref, q2_ref, q3_ref, out_ref):
  parts = [_bf16_word_pack(r[...].T) for r in (q0_ref, q1_ref, q2_ref, q3_ref)]
  out_ref[...] = jnp.concatenate(parts, axis=1)


def _pack_text_bf16(table, n_rows):
  nblk = (n_rows + PACK_BLK - 1) // PACK_BLK
  assert nblk % 4 == 0, "quarter block count required"
  pad = nblk * PACK_BLK
  qblk = nblk // 4
  t_t = table.T
  out = pl.pallas_call(
      _pack_text_body,
      grid=(qblk,),
      in_specs=[
          pl.BlockSpec((EMB, PACK_BLK), lambda i, q=q: (0, q * qblk + i))
          for q in range(4)
      ],
      out_specs=pl.BlockSpec((PACK_BLK, 2 * EMB), lambda i: (i, 0)),
      out_shape=jax.ShapeDtypeStruct((pad // 4, 2 * EMB), jnp.float32),
  )(t_t, t_t, t_t, t_t)
  return out, pad // 4


def _pack_table(table, n_rows):
  nblk = (n_rows + PACK_BLK - 1) // PACK_BLK
  assert nblk % 2 == 0, "even block count required"
  pad = nblk * PACK_BLK
  half_blk = nblk // 2
  t_t = table.T
  out = pl.pallas_call(
      _pack_body,
      grid=(half_blk,),
      in_specs=[
          pl.BlockSpec((EMB, PACK_BLK), lambda i: (0, i)),
          pl.BlockSpec((EMB, PACK_BLK), lambda i: (0, half_blk + i)),
      ],
      out_specs=pl.BlockSpec((PACK_BLK, 2 * EMB), lambda i: (i, 0)),
      out_shape=jax.ShapeDtypeStruct((pad // 2, 2 * EMB), jnp.float32),
  )(t_t, t_t)
  return out, pad // 2


def _remap_idx(idx, half):
  return jnp.where(idx < half, 2 * idx, 2 * (idx - half) + 1).astype(jnp.int32)


def _sc_text(tok2d, text_packed):
  rows4 = text_packed.shape[0] * 4
  text_lin = text_packed.reshape(rows4, EMB // 2)
  mesh = plsc.VectorSubcoreMesh(core_axis_name="c", subcore_axis_name="s")

  @functools.partial(
      pl.kernel,
      mesh=mesh,
      compiler_params=pltpu.CompilerParams(use_tc_tiling_on_sc=False,
                                           needs_layout_passes=False),
      out_type=jax.ShapeDtypeStruct((B, 2 * EMB), jnp.float32),
      scratch_types=[
          pltpu.VMEM((NCH, CHUNK), jnp.int32),
          pltpu.VMEM((CHUNK, EMB // 2), jnp.float32),
          pltpu.VMEM((CHUNK, EMB // 2), jnp.float32),
          pltpu.VMEM((BPW, 2 * EMB), jnp.float32),
          pltpu.SemaphoreType.DMA,
          pltpu.SemaphoreType.DMA,
      ],
  )
  def sc_kernel(tok_hbm, table_hbm, text_out_hbm,
                tok_idx_v, gbuf0, gbuf1, acc, sem_t0, sem_t1):
    wid = lax.axis_index("s") * NC + lax.axis_index("c")
    base = wid * BPW
    gbufs = (gbuf0, gbuf1)
    sems = (sem_t0, sem_t1)

    pltpu.sync_copy(tok_hbm.at[pl.ds(wid * NCH, NCH)], tok_idx_v)
    pltpu.async_copy(table_hbm.at[tok_idx_v.at[0]], gbufs[0], sems[0])

    @pl.loop(0, NCH, step=2)
    def chunk_loop(c):
      for b in range(2):
        cc = c + b
        nxt = (b + 1) % 2

        @pl.when(cc + 1 < NCH)
        def _():
          pltpu.async_copy(table_hbm.at[tok_idx_v.at[cc + 1]],
                           gbufs[nxt], sems[nxt])

        pltpu.make_async_copy(table_hbm.at[tok_idx_v.at[0]],
                              gbufs[b], sems[b]).wait()
        gbuf = gbufs[b]
        for i in range(ROWS_PER_CHUNK):
          acc_a = [None, None]
          acc_b = [None, None]
          for j in range(SEQ):
            r = i * SEQ + j
            for l in range(2):
              w = gbuf[r, pl.ds(l * 16, 16)]
              wb = plsc.bitcast(w, jnp.bfloat16)
              a, bb = plsc.unpack(wb, format=plsc.PackFormat.INTERLEAVED)
              acc_a[l] = a if acc_a[l] is None else acc_a[l] + a
              acc_b[l] = bb if acc_b[l] is None else acc_b[l] + bb
          row = cc * ROWS_PER_CHUNK + i
          acc[row, pl.ds(0, 16)] = acc_a[0]
          acc[row, pl.ds(16, 16)] = acc_b[0]
          acc[row, pl.ds(32, 16)] = acc_a[1]
          acc[row, pl.ds(48, 16)] = acc_b[1]

    pltpu.sync_copy(acc, text_out_hbm.at[pl.ds(base, BPW)])

  return sc_kernel(tok2d, text_lin)


def _sc_item(item_idx, item_pad):
  mesh = plsc.VectorSubcoreMesh(core_axis_name="c", subcore_axis_name="s")

  @functools.partial(
      pl.kernel,
      mesh=mesh,
      compiler_params=pltpu.CompilerParams(use_tc_tiling_on_sc=False),
      out_type=jax.ShapeDtypeStruct((B, 2 * EMB), jnp.float32),
      scratch_types=[
          pltpu.VMEM((BPW,), jnp.int32),
          pltpu.VMEM((BPW, 2 * EMB), jnp.float32),
          pltpu.SemaphoreType.DMA,
      ],
  )
  def sc_kernel(idx_hbm, table_hbm, out_hbm, idx_v, rows_v, sem):
    wid = lax.axis_index("s") * NC + lax.axis_index("c")
    base = wid * BPW
    pltpu.sync_copy(idx_hbm.at[pl.ds(base, BPW)], idx_v)
    pltpu.async_copy(table_hbm.at[idx_v], rows_v, sem).wait()
    pltpu.sync_copy(rows_v, out_hbm.at[pl.ds(base, BPW)])

  return sc_kernel(item_idx, item_pad)


def _tc_img_body(img_ref, wimg_ref, bimg_ref, out_ref):
  out_ref[...] = jnp.dot(img_ref[...], wimg_ref[...],
                         preferred_element_type=jnp.float32) + bimg_ref[...]


def _tc_img(image_embedding, W_img, b_img):
  TILE = 512
  return pl.pallas_call(
      _tc_img_body,
      grid=(B // TILE,),
      in_specs=[
          pl.BlockSpec((TILE, IMG_D), lambda i: (i, 0)),
          pl.BlockSpec((IMG_D, EMB), lambda i: (0, 0)),
          pl.BlockSpec((1, EMB), lambda i: (0, 0)),
      ],
      out_specs=pl.BlockSpec((TILE, EMB), lambda i: (i, 0)),
      out_shape=jax.ShapeDtypeStruct((B, EMB), jnp.float32),
  )(image_embedding, W_img, b_img)


def _tc_final_body(item_ref, text_ref, img_ref, wproj_ref, bproj_ref,
                   out_ref):
  w = wproj_ref[...]
  part = jnp.dot(item_ref[:, :EMB], w[:EMB, :],
                 preferred_element_type=jnp.float32)
  part += jnp.dot(text_ref[:, :EMB] * (1.0 / SEQ), w[EMB:2 * EMB, :],
                  preferred_element_type=jnp.float32)
  part += jnp.dot(img_ref[...], w[2 * EMB:, :],
                  preferred_element_type=jnp.float32)
  out_ref[...] = part + bproj_ref[...]


def _tc_final(item_vec, text_sum, img_vec, W_proj, b_proj):
  TILE = 512
  return pl.pallas_call(
      _tc_final_body,
      grid=(B // TILE,),
      in_specs=[
          pl.BlockSpec((TILE, 2 * EMB), lambda i: (i, 0)),
          pl.BlockSpec((TILE, 2 * EMB), lambda i: (i, 0)),
          pl.BlockSpec((TILE, EMB), lambda i: (i, 0)),
          pl.BlockSpec((3 * EMB, EMB), lambda i: (0, 0)),
          pl.BlockSpec((1, EMB), lambda i: (0, 0)),
      ],
      out_specs=pl.BlockSpec((TILE, EMB), lambda i: (i, 0)),
      out_shape=jax.ShapeDtypeStruct((B, EMB), jnp.float32),
  )(item_vec, text_sum, img_vec, W_proj, b_proj)


@jax.jit
def kernel(item_id, text_tokens, image_embedding, item_table, text_table,
           W_img, b_img, W_proj, b_proj):
  text_packed, text_q = _pack_text_bf16(text_table, N_TOKENS)
  tok_mapped = (4 * (text_tokens % text_q)
                + text_tokens // text_q).astype(jnp.int32)
  tok2d = tok_mapped.reshape(B * SEQ // CHUNK, CHUNK)
  text_sum = _sc_text(tok2d, text_packed)

  perm = jnp.concatenate([
      jnp.arange(0, 16), jnp.arange(32, 48),
      jnp.arange(16, 32), jnp.arange(48, 64)]) + EMB
  W_proj = jnp.concatenate(
      [W_proj[:EMB], W_proj[perm], W_proj[2 * EMB:]], axis=0)

  item_pad = jnp.pad(item_table, ((0, 7), (0, EMB)))
  item_vec = _sc_item(item_id, item_pad)

  img_vec = _tc_img(image_embedding, W_img, b_img.reshape(1, EMB))
  return _tc_final(item_vec, text_sum, img_vec, W_proj,
                   b_proj.reshape(1, EMB))
